# baseline (speedup 1.0000x reference)
import functools

import jax
import jax.numpy as jnp
from jax.experimental import pallas as pl
from jax.experimental.pallas import tpu as pltpu

_HEADS = 12


def _kv_proj_kernel(xy_ref, w_ref, kv_ref):
    kv_ref[0, 0] = jnp.dot(
        xy_ref[0], w_ref[0], preferred_element_type=jnp.float32
    )


def _mono(bits):
    return jnp.where(bits >= 0, bits, bits ^ jnp.int32(0x7FFFFFFF))


def _inv_mono(m):
    bits = jnp.where(m >= 0, m, m ^ jnp.int32(0x7FFFFFFF))
    return jax.lax.bitcast_convert_type(bits, jnp.float32)


def _bisect_kth(dsub, kk):
    nfull = dsub.shape[1]
    dmin = jnp.min(dsub, axis=1, keepdims=True)
    dmax = jnp.max(dsub, axis=1, keepdims=True)
    mmin = _mono(jax.lax.bitcast_convert_type(dmin, jnp.int32))
    mmax = _mono(jax.lax.bitcast_convert_type(dmax, jnp.int32))

    def count(m):
        return jnp.sum((dsub >= _inv_mono(m)).astype(jnp.int32), axis=1,
                       keepdims=True)

    lo0 = mmax - jnp.int32(1 << 25)
    lo0 = jnp.where(jnp.logical_or(lo0 > mmax, lo0 < mmin), mmin, lo0)
    cnt0 = count(lo0)
    ok0 = cnt0 >= kk
    lo = jnp.where(ok0, lo0, mmin)
    clo = jnp.where(ok0, cnt0, jnp.int32(nfull))
    hi = jnp.where(ok0,
                   jnp.where(cnt0 == kk, lo0 + 1, mmax + 1),
                   lo0)
    chi = jnp.where(ok0, jnp.int32(0), cnt0)

    def cond(carry):
        it, lo, clo, hi, chi = carry
        return jnp.logical_and(it < 48, jnp.max(hi - lo) > 1)

    def update(mid, lo, clo, hi, chi):
        cnt = count(mid)
        ge = cnt >= kk
        eq = cnt == kk
        lo2 = jnp.where(ge, mid, lo)
        clo2 = jnp.where(ge, cnt, clo)
        hi2 = jnp.where(eq, mid + 1, jnp.where(ge, hi, mid))
        chi2 = jnp.where(ge, chi, cnt)
        return lo2, clo2, hi2, chi2

    def istep(lo, clo, hi, chi):
        t = (clo - kk).astype(jnp.float32) / jnp.maximum(
            clo - chi, 1).astype(jnp.float32)
        lof = lo.astype(jnp.float32)
        midf = lof + t * (hi.astype(jnp.float32) - lof)
        mid = jnp.clip(midf, -2.1e9, 2.1e9).astype(jnp.int32)
        mid = jnp.clip(mid, lo + 1, hi - 1)
        return update(mid, lo, clo, hi, chi)

    def bstep(lo, clo, hi, chi):
        mid = (lo >> 1) + (hi >> 1) + (lo & hi & 1)
        return update(mid, lo, clo, hi, chi)

    def body(carry):
        it, lo, clo, hi, chi = carry
        lo, clo, hi, chi = istep(lo, clo, hi, chi)
        lo, clo, hi, chi = bstep(lo, clo, hi, chi)
        return it + 2, lo, clo, hi, chi

    _, lo, _, _, _ = jax.lax.while_loop(cond, body, (0, lo, clo, hi, chi))
    return _inv_mono(lo), dmax


def _attn_kernel(srcq_ref, wq_ref, k_ref, v_ref, wout_ref, b_ref, out_ref,
                 *, kk, scale, sub):
    h = pl.program_id(2)
    q = jnp.dot(srcq_ref[0], wq_ref[0], preferred_element_type=jnp.float32)
    k = k_ref[0, 0]
    v = v_ref[0, 0]
    dots = jax.lax.dot_general(
        q, k, (((1,), (1,)), ((), ())), preferred_element_type=jnp.float32
    ) * scale

    tf, dmax = _bisect_kth(dots, kk)

    p = jnp.where(dots >= tf, jnp.exp(dots - dmax), 0.0)
    s = jnp.sum(p, axis=1, keepdims=True)
    o = jnp.dot(p, v, preferred_element_type=jnp.float32) / s
    contrib = jnp.dot(o, wout_ref[...], preferred_element_type=jnp.float32)

    @pl.when(h == 0)
    def _():
        out_ref[...] = (contrib + b_ref[...])[None]

    @pl.when(h != 0)
    def _():
        out_ref[...] += contrib[None]


def _inter_attention(x, y, Wq, Wkv, Wout, b_out, heads, iblk, interpret=False):
    _, n, dim = x.shape
    inner = Wq.shape[1]
    dh = inner // heads
    scale = dh ** -0.5
    kk = max(n // 4, 1)

    xy = jnp.concatenate([x, y], axis=0)
    b2 = b_out.reshape(1, dim)
    wq3 = Wq.reshape(dim, heads, dh).transpose(1, 0, 2)
    wkv3 = Wkv.reshape(dim, 2 * heads, dh).transpose(1, 0, 2)

    kv = pl.pallas_call(
        _kv_proj_kernel,
        grid=(2, 2 * heads),
        in_specs=[
            pl.BlockSpec((1, n, dim), lambda s, c: (s, 0, 0)),
            pl.BlockSpec((1, dim, dh), lambda s, c: (c, 0, 0)),
        ],
        out_specs=pl.BlockSpec((1, 1, n, dh), lambda s, c: (1 - s, c, 0, 0)),
        out_shape=jax.ShapeDtypeStruct((2, 2 * heads, n, dh), jnp.float32),
        compiler_params=pltpu.CompilerParams(
            dimension_semantics=("arbitrary", "arbitrary"),
        ),
        interpret=interpret,
    )(xy, wkv3)

    nblk = n // iblk
    out = pl.pallas_call(
        functools.partial(_attn_kernel, kk=kk, scale=scale, sub=64),
        grid=(2, nblk, heads),
        in_specs=[
            pl.BlockSpec((1, iblk, dim), lambda d, i, h: (d, i, 0)),
            pl.BlockSpec((1, dim, dh), lambda d, i, h: (h, 0, 0)),
            pl.BlockSpec((1, 1, n, dh), lambda d, i, h: (d, h, 0, 0)),
            pl.BlockSpec((1, 1, n, dh), lambda d, i, h: (d, heads + h, 0, 0)),
            pl.BlockSpec((dh, dim), lambda d, i, h: (h, 0)),
            pl.BlockSpec((1, dim), lambda d, i, h: (0, 0)),
        ],
        out_specs=pl.BlockSpec((1, iblk, dim), lambda d, i, h: (d, i, 0)),
        out_shape=jax.ShapeDtypeStruct((2, n, dim), jnp.float32),
        compiler_params=pltpu.CompilerParams(
            dimension_semantics=("parallel", "parallel", "arbitrary"),
        ),
        interpret=interpret,
    )(xy, wq3, kv, kv, Wout, b2)

    return out[0:1], out[1:2]


def kernel(x, y, Wq, Wkv, Wout, b_out):
    return _inter_attention(x, y, Wq, Wkv, Wout, b_out, heads=_HEADS, iblk=1024)


# speedup vs baseline: 1.1628x; 1.1628x over previous
import functools

import jax
import jax.numpy as jnp
from jax.experimental import pallas as pl
from jax.experimental.pallas import tpu as pltpu

_HEADS = 12


def _kv_proj_kernel(xy_ref, w_ref, kv_ref):
    kv_ref[0, 0] = jnp.dot(
        xy_ref[0], w_ref[0], preferred_element_type=jnp.float32
    )


def _mono(bits):
    return jnp.where(bits >= 0, bits, bits ^ jnp.int32(0x7FFFFFFF))


def _inv_mono(m):
    bits = jnp.where(m >= 0, m, m ^ jnp.int32(0x7FFFFFFF))
    return jax.lax.bitcast_convert_type(bits, jnp.float32)


def _bisect_kth(dsub, kk):
    dmin = jnp.min(dsub, axis=1, keepdims=True)
    dmax = jnp.max(dsub, axis=1, keepdims=True)
    mmin = _mono(jax.lax.bitcast_convert_type(dmin, jnp.int32))
    mmax = _mono(jax.lax.bitcast_convert_type(dmax, jnp.int32))

    lo0 = mmax - jnp.int32(1 << 25)
    lo0 = jnp.where(jnp.logical_or(lo0 > mmax, lo0 < mmin), mmin, lo0)
    cnt0 = jnp.sum((dsub >= _inv_mono(lo0)).astype(jnp.int32), axis=1,
                   keepdims=True)
    ok0 = cnt0 >= kk
    lo = jnp.where(ok0, lo0, mmin)
    hi = jnp.where(ok0,
                   jnp.where(cnt0 == kk, lo0 + 1, mmax + 1),
                   lo0)

    def cond(carry):
        it, lo, hi = carry
        return jnp.logical_and(it < 32, jnp.max(hi - lo) > 1)

    def step(lo, hi):
        mid = (lo >> 1) + (hi >> 1) + (lo & hi & 1)
        midf = _inv_mono(mid)
        cnt = jnp.sum((dsub >= midf).astype(jnp.int32), axis=1,
                      keepdims=True)
        ge = cnt >= kk
        eq = cnt == kk
        lo2 = jnp.where(ge, mid, lo)
        hi2 = jnp.where(eq, mid + 1, jnp.where(ge, hi, mid))
        return lo2, hi2

    def body(carry):
        it, lo, hi = carry
        lo, hi = step(lo, hi)
        lo, hi = step(lo, hi)
        lo, hi = step(lo, hi)
        return it + 3, lo, hi

    _, lo, _ = jax.lax.while_loop(cond, body, (0, lo, hi))
    return _inv_mono(lo), dmax


def _attn_kernel(srcq_ref, wq_ref, k_ref, v_ref, wout_ref, b_ref, out_ref,
                 *, kk, scale, sub):
    h = pl.program_id(2)
    q = jnp.dot(srcq_ref[0], wq_ref[0], preferred_element_type=jnp.float32)
    k = k_ref[0, 0]
    v = v_ref[0, 0]
    dots = jax.lax.dot_general(
        q, k, (((1,), (1,)), ((), ())), preferred_element_type=jnp.float32
    ) * scale

    tf, dmax = _bisect_kth(dots, kk)

    p = jnp.where(dots >= tf, jnp.exp(dots - dmax), 0.0)
    s = jnp.sum(p, axis=1, keepdims=True)
    o = jnp.dot(p, v, preferred_element_type=jnp.float32) / s
    contrib = jnp.dot(o, wout_ref[...], preferred_element_type=jnp.float32)

    @pl.when(h == 0)
    def _():
        out_ref[...] = (contrib + b_ref[...])[None]

    @pl.when(h != 0)
    def _():
        out_ref[...] += contrib[None]


def _inter_attention(x, y, Wq, Wkv, Wout, b_out, heads, iblk, interpret=False):
    _, n, dim = x.shape
    inner = Wq.shape[1]
    dh = inner // heads
    scale = dh ** -0.5
    kk = max(n // 4, 1)

    xy = jnp.concatenate([x, y], axis=0)
    b2 = b_out.reshape(1, dim)
    wq3 = Wq.reshape(dim, heads, dh).transpose(1, 0, 2)
    wkv3 = Wkv.reshape(dim, 2 * heads, dh).transpose(1, 0, 2)

    kv = pl.pallas_call(
        _kv_proj_kernel,
        grid=(2, 2 * heads),
        in_specs=[
            pl.BlockSpec((1, n, dim), lambda s, c: (s, 0, 0)),
            pl.BlockSpec((1, dim, dh), lambda s, c: (c, 0, 0)),
        ],
        out_specs=pl.BlockSpec((1, 1, n, dh), lambda s, c: (1 - s, c, 0, 0)),
        out_shape=jax.ShapeDtypeStruct((2, 2 * heads, n, dh), jnp.float32),
        compiler_params=pltpu.CompilerParams(
            dimension_semantics=("arbitrary", "arbitrary"),
        ),
        interpret=interpret,
    )(xy, wkv3)

    nblk = n // iblk
    out = pl.pallas_call(
        functools.partial(_attn_kernel, kk=kk, scale=scale, sub=64),
        grid=(2, nblk, heads),
        in_specs=[
            pl.BlockSpec((1, iblk, dim), lambda d, i, h: (d, i, 0)),
            pl.BlockSpec((1, dim, dh), lambda d, i, h: (h, 0, 0)),
            pl.BlockSpec((1, 1, n, dh), lambda d, i, h: (d, h, 0, 0)),
            pl.BlockSpec((1, 1, n, dh), lambda d, i, h: (d, heads + h, 0, 0)),
            pl.BlockSpec((dh, dim), lambda d, i, h: (h, 0)),
            pl.BlockSpec((1, dim), lambda d, i, h: (0, 0)),
        ],
        out_specs=pl.BlockSpec((1, iblk, dim), lambda d, i, h: (d, i, 0)),
        out_shape=jax.ShapeDtypeStruct((2, n, dim), jnp.float32),
        compiler_params=pltpu.CompilerParams(
            dimension_semantics=("parallel", "parallel", "arbitrary"),
        ),
        interpret=interpret,
    )(xy, wq3, kv, kv, Wout, b2)

    return out[0:1], out[1:2]


def kernel(x, y, Wq, Wkv, Wout, b_out):
    return _inter_attention(x, y, Wq, Wkv, Wout, b_out, heads=_HEADS, iblk=512)
